# baseline (speedup 1.0000x reference)
import functools

import jax
import jax.numpy as jnp
from jax.experimental import pallas as pl
from jax.experimental.pallas import tpu as pltpu


def _bn(h, g, b):
    m = jnp.mean(h, axis=0)
    v = jnp.var(h, axis=0)
    return (h - m) / jnp.sqrt(v + 1e-5) * g + b


def _gin(x, src, dst, W1, b1, W2, b2):
    agg = jnp.zeros_like(x).at[dst].add(x[src])
    h = x + agg
    h = jnp.maximum(h @ W1 + b1, 0.0) @ W2 + b2
    return h


def _head_kernel(p_ref, fW1_ref, fb1_ref, fW2_ref, fb2_ref, oW_ref, ob_ref, o_ref):
    p = p_ref[...]
    p = jnp.maximum(p @ fW1_ref[...] + fb1_ref[...], 0.0)
    p = jnp.maximum(p @ fW2_ref[...] + fb2_ref[...], 0.0)
    o_ref[...] = p @ oW_ref[...] + ob_ref[...]


def kernel(x, edge_index, batch,
           W1, b1, W2, b2, g1, be1,
           W3, b3, W4, b4, g2, be2,
           W5, b5, W6, b6, g3, be3,
           fW1, fb1, fW2, fb2, oW, ob):
    src, dst = edge_index[0], edge_index[1]
    h = jnp.maximum(_gin(x, src, dst, W1, b1, W2, b2), 0.0)
    h = _bn(h, g1, be1)
    h = jnp.maximum(_gin(h, src, dst, W3, b3, W4, b4), 0.0)
    h = _bn(h, g2, be2)
    h = jnp.maximum(_gin(h, src, dst, W5, b5, W6, b6), 0.0)
    h = _bn(h, g3, be3)
    G = 64
    p = jax.ops.segment_sum(h, batch, num_segments=G)
    out = pl.pallas_call(
        _head_kernel,
        out_shape=jax.ShapeDtypeStruct((G, 2), jnp.float32),
    )(p, fW1, fb1, fW2, fb2, oW, ob)
    return out


# speedup vs baseline: 11.0619x; 11.0619x over previous
import functools

import jax
import jax.numpy as jnp
from jax import lax
from jax.experimental import pallas as pl
from jax.experimental.pallas import tpu as pltpu
from jax.experimental.pallas import tpu_sc as plsc

_N = 10000
_E = 640000
_G = 64

_NC, _NS, _L = 2, 16, 16
_NW = _NC * _NS
_CHUNK = 128


def _make_sc_scatter(feat):
    epw = _E // _NW
    n_full = epw // _CHUNK
    rem = epw - n_full * _CHUNK
    assert n_full % 2 == 0 and rem % 8 == 0 and feat % _L == 0
    base_rows = (_N // (_NS * 8)) * 8
    last_rows = _N - base_rows * (_NS - 1)
    zfull, ztail = base_rows // _CHUNK, base_rows % _CHUNK
    zfull_l, ztail_l = last_rows // _CHUNK, last_rows % _CHUNK

    mesh = plsc.VectorSubcoreMesh(core_axis_name="c", subcore_axis_name="s")

    @functools.partial(
        pl.kernel, mesh=mesh,
        out_type=jax.ShapeDtypeStruct((_NC, _N, feat), jnp.float32),
        compiler_params=pltpu.CompilerParams(use_tc_tiling_on_sc=False),
        scratch_types=[
            pltpu.VMEM((_CHUNK,), jnp.int32), pltpu.VMEM((_CHUNK,), jnp.int32),
            pltpu.VMEM((_CHUNK,), jnp.int32), pltpu.VMEM((_CHUNK,), jnp.int32),
            pltpu.VMEM((_CHUNK, feat), jnp.float32),
            pltpu.VMEM((_CHUNK, feat), jnp.float32),
            pltpu.VMEM((rem,), jnp.int32), pltpu.VMEM((rem,), jnp.int32),
            pltpu.VMEM((rem, feat), jnp.float32),
            pltpu.VMEM_SHARED((_N, feat), jnp.float32),
            pltpu.SemaphoreType.DMA, pltpu.SemaphoreType.DMA,
        ],
    )
    def sc(x_hbm, src_hbm, dst_hbm, out_hbm,
           sidx0, didx0, sidx1, didx1, rows0, rows1,
           sidx2, didx2, rows2, acc, sem0, sem1):
        c = lax.axis_index("c")
        s = lax.axis_index("s")
        wid = s * _NC + c
        sidx = (sidx0, sidx1)
        didx = (didx0, didx1)
        rows = (rows0, rows1)
        sems = (sem0, sem1)

        def zrow(i, _):
            def zcol(j, _):
                rows0[i, pl.ds(j * _L, _L)] = jnp.zeros((_L,), jnp.float32)
                return 0
            return lax.fori_loop(0, feat // _L, zcol, 0)
        lax.fori_loop(0, _CHUNK, zrow, 0)

        r0 = s * base_rows
        nz = jnp.where(s == _NS - 1, zfull_l, zfull)

        def zacc(i, _):
            pltpu.sync_copy(rows0, acc.at[pl.ds(r0 + i * _CHUNK, _CHUNK)])
            return 0
        lax.fori_loop(0, nz, zacc, 0)
        if ztail:
            @pl.when(s < _NS - 1)
            def _():
                pltpu.sync_copy(rows0.at[pl.ds(0, ztail)],
                                acc.at[pl.ds(r0 + zfull * _CHUNK, ztail)])
        if ztail_l:
            @pl.when(s == _NS - 1)
            def _():
                pltpu.sync_copy(rows0.at[pl.ds(0, ztail_l)],
                                acc.at[pl.ds(r0 + zfull_l * _CHUNK, ztail_l)])
        plsc.subcore_barrier()

        ebase = wid * epw

        def start(b, off):
            pltpu.sync_copy(src_hbm.at[pl.ds(off, _CHUNK)], sidx[b])
            pltpu.sync_copy(dst_hbm.at[pl.ds(off, _CHUNK)], didx[b])
            pltpu.async_copy(x_hbm.at[sidx[b]], rows[b], sems[b])

        def finish(b):
            pltpu.make_async_copy(x_hbm.at[sidx[b]], rows[b], sems[b]).wait()
            pltpu.sync_copy(rows[b], acc.at[didx[b]], add=True)

        start(0, ebase)

        def pair(kk, _):
            k = kk * 2
            start(1, ebase + (k + 1) * _CHUNK)
            finish(0)

            @pl.when(k + 2 < n_full)
            def _():
                start(0, ebase + (k + 2) * _CHUNK)
            finish(1)
            return 0
        lax.fori_loop(0, n_full // 2, pair, 0)

        if rem:
            off = ebase + n_full * _CHUNK
            pltpu.sync_copy(src_hbm.at[pl.ds(off, rem)], sidx2)
            pltpu.sync_copy(dst_hbm.at[pl.ds(off, rem)], didx2)
            pltpu.async_copy(x_hbm.at[sidx2], rows2, sem0).wait()
            pltpu.sync_copy(rows2, acc.at[didx2], add=True)

        plsc.subcore_barrier()

        @pl.when(s < _NS - 1)
        def _():
            pltpu.sync_copy(acc.at[pl.ds(r0, base_rows)],
                            out_hbm.at[c, pl.ds(r0, base_rows)])

        @pl.when(s == _NS - 1)
        def _():
            pltpu.sync_copy(acc.at[pl.ds(r0, last_rows)],
                            out_hbm.at[c, pl.ds(r0, last_rows)])

    return sc


_sc_scatter = {f: _make_sc_scatter(f) for f in (128, 112, 96, 64)}

_R = 2000
_NB = _N // _R


def _make_layer(fins, hmid, hout, outs):
    k = len(fins)
    covered = sum(outs)

    def body(*refs):
        x_refs = refs[0:k]
        agg_refs = refs[k:2 * k]
        W1_refs = refs[2 * k:3 * k]
        b1_ref, W2_ref, b2_ref, g_ref, be_ref = refs[3 * k:3 * k + 5]
        o_refs = refs[3 * k + 5:3 * k + 5 + len(outs)]
        h_all, stats = refs[3 * k + 5 + len(outs):]
        i = pl.program_id(0)

        @pl.when(i == 0)
        def _():
            stats[...] = jnp.zeros_like(stats)

        @pl.when(i < _NB)
        def _():
            a1 = b1_ref[...] + jnp.zeros((_R, hmid), jnp.float32)
            for xr, ar, wr in zip(x_refs, agg_refs, W1_refs):
                t = xr[...] + ar[0] + ar[1]
                a1 = a1 + jnp.dot(t, wr[...],
                                  preferred_element_type=jnp.float32)
            a1 = jnp.maximum(a1, 0.0)
            a2 = jnp.dot(a1, W2_ref[...], preferred_element_type=jnp.float32)
            a2 = jnp.maximum(a2 + b2_ref[...], 0.0)
            h_all[pl.ds(i * _R, _R)] = a2
            stats[0:1, :] = stats[0:1, :] + jnp.sum(a2, axis=0, keepdims=True)
            stats[1:2, :] = stats[1:2, :] + jnp.sum(a2 * a2, axis=0,
                                                    keepdims=True)

        @pl.when(i == _NB)
        def _():
            m = stats[0:1, :] / _N
            v = stats[1:2, :] / _N - m * m
            a = g_ref[...] / jnp.sqrt(v + 1e-5)
            stats[2:3, :] = a
            stats[3:4, :] = be_ref[...] - m * a

        @pl.when(i >= _NB)
        def _():
            j = i - _NB
            hb = h_all[pl.ds(j * _R, _R)] * stats[2:3, :] + stats[3:4, :]
            col = 0
            for o_ref, w in zip(o_refs, outs):
                if col + w <= hout:
                    o_ref[...] = hb[:, col:col + w]
                else:
                    o_ref[...] = jnp.concatenate(
                        [hb[:, col:hout],
                         jnp.zeros((_R, col + w - hout), jnp.float32)], axis=1)
                col += w

    in_specs = (
        [pl.BlockSpec((_R, f), lambda i: (i % _NB, 0)) for f in fins]
        + [pl.BlockSpec((2, _R, f), lambda i: (0, i % _NB, 0)) for f in fins]
        + [pl.BlockSpec((f, hmid), lambda i: (0, 0)) for f in fins]
        + [pl.BlockSpec((1, hmid), lambda i: (0, 0)),
           pl.BlockSpec((hmid, hout), lambda i: (0, 0)),
           pl.BlockSpec((1, hout), lambda i: (0, 0)),
           pl.BlockSpec((1, hout), lambda i: (0, 0)),
           pl.BlockSpec((1, hout), lambda i: (0, 0))]
    )
    return pl.pallas_call(
        body,
        grid=(2 * _NB,),
        in_specs=in_specs,
        out_specs=[pl.BlockSpec((_R, w), lambda i: (i % _NB, 0))
                   for w in outs],
        out_shape=[jax.ShapeDtypeStruct((_N, w), jnp.float32) for w in outs],
        scratch_shapes=[
            pltpu.VMEM((_N, hout), jnp.float32),
            pltpu.VMEM((8, hout), jnp.float32),
        ],
    )


_layer1 = _make_layer([128], 198, 198, [112, 96])
_layer2 = _make_layer([112, 96], 64, 64, [64])
_layer3 = _make_layer([64], 32, 32, [32])


def _head_body(h_ref, b_ref, fW1_ref, fb1_ref, fW2_ref, fb2_ref,
               oW_ref, ob_ref, o_ref, pacc):
    i = pl.program_id(0)

    @pl.when(i == 0)
    def _():
        pacc[...] = jnp.zeros_like(pacc)

    b = b_ref[0]
    onehot = (lax.broadcasted_iota(jnp.int32, (_G, 1), 0) == b)
    onehot = onehot.astype(jnp.float32)
    pacc[...] = pacc[...] + lax.dot_general(
        onehot, h_ref[...], (((1,), (0,)), ((), ())),
        preferred_element_type=jnp.float32,
        precision=lax.Precision.HIGHEST)

    @pl.when(i == _NB - 1)
    def _():
        p = pacc[...]
        p = jnp.maximum(
            jnp.dot(p, fW1_ref[...], preferred_element_type=jnp.float32)
            + fb1_ref[...], 0.0)
        p = jnp.maximum(
            jnp.dot(p, fW2_ref[...], preferred_element_type=jnp.float32)
            + fb2_ref[...], 0.0)
        o_ref[...] = jnp.dot(
            p, oW_ref[...], preferred_element_type=jnp.float32) + ob_ref[...]


_head = pl.pallas_call(
    _head_body,
    grid=(_NB,),
    in_specs=[
        pl.BlockSpec((_R, 32), lambda i: (i, 0)),
        pl.BlockSpec((1, 1, _R), lambda i: (i, 0, 0)),
        pl.BlockSpec((32, 16), lambda i: (0, 0)),
        pl.BlockSpec((1, 16), lambda i: (0, 0)),
        pl.BlockSpec((16, 8), lambda i: (0, 0)),
        pl.BlockSpec((1, 8), lambda i: (0, 0)),
        pl.BlockSpec((8, 2), lambda i: (0, 0)),
        pl.BlockSpec((1, 2), lambda i: (0, 0)),
    ],
    out_specs=pl.BlockSpec((_G, 2), lambda i: (0, 0)),
    out_shape=jax.ShapeDtypeStruct((_G, 2), jnp.float32),
    scratch_shapes=[pltpu.VMEM((_G, 32), jnp.float32)],
)


def kernel(x, edge_index, batch,
           W1, b1, W2, b2, g1, be1,
           W3, b3, W4, b4, g2, be2,
           W5, b5, W6, b6, g3, be3,
           fW1, fb1, fW2, fb2, oW, ob):
    src = edge_index[0]
    dst = edge_index[1]
    xp = jnp.pad(x, ((0, 0), (0, 128 - 114)))
    W1p = jnp.pad(W1, ((0, 128 - 114), (0, 0)))
    W3a = W3[:112]
    W3b = jnp.pad(W3[112:], ((0, 96 - (198 - 112)), (0, 0)))

    agg0 = _sc_scatter[128](xp, src, dst)
    h1a, h1b = _layer1(xp, agg0, W1p, b1[None], W2, b2[None],
                       g1[None], be1[None])
    aggA = _sc_scatter[112](h1a, src, dst)
    aggB = _sc_scatter[96](h1b, src, dst)
    h2, = _layer2(h1a, h1b, aggA, aggB, W3a, W3b, b3[None], W4, b4[None],
                  g2[None], be2[None])
    agg2 = _sc_scatter[64](h2, src, dst)
    h3, = _layer3(h2, agg2, W5, b5[None], W6, b6[None], g3[None], be3[None])
    batch3 = batch.reshape(_NB, 1, _R)
    return _head(h3, batch3, fW1, fb1[None], fW2, fb2[None], oW, ob[None])


# speedup vs baseline: 16.1705x; 1.4618x over previous
import functools

import jax
import jax.numpy as jnp
from jax import lax
from jax.experimental import pallas as pl
from jax.experimental.pallas import tpu as pltpu
from jax.experimental.pallas import tpu_sc as plsc

_N = 10000
_E = 640000
_G = 64

_NC, _NS, _L = 2, 16, 16
_NW = _NC * _NS
_CHUNK = 128


_NCHUNK = _E // _CHUNK
_CPW = _NCHUNK // _NW
_XTRA = _NCHUNK - _CPW * _NW

_PIPE = {128: (2, 52), 112: (3, 39), 96: (4, 52), 64: (4, 52)}


def _make_sc_scatter(feat):
    ring, iblk = _PIPE[feat]
    pf = {2: 1, 3: 2, 4: 2}[ring]
    assert _CPW % iblk == 0 and iblk % ring == 0 and feat % _L == 0
    nblk = _CPW // iblk
    base_rows = (_N // (_NS * 8)) * 8
    last_rows = _N - base_rows * (_NS - 1)
    zfull, ztail = base_rows // _CHUNK, base_rows % _CHUNK
    zfull_l, ztail_l = last_rows // _CHUNK, last_rows % _CHUNK

    mesh = plsc.VectorSubcoreMesh(core_axis_name="c", subcore_axis_name="s")

    @functools.partial(
        pl.kernel, mesh=mesh,
        out_type=jax.ShapeDtypeStruct((_NC, _N, feat), jnp.float32),
        compiler_params=pltpu.CompilerParams(use_tc_tiling_on_sc=False),
        scratch_types=(
            [pltpu.VMEM((iblk, _CHUNK), jnp.int32),
             pltpu.VMEM((iblk, _CHUNK), jnp.int32)]
            + [pltpu.VMEM((_CHUNK, feat), jnp.float32) for _ in range(ring)]
            + [pltpu.VMEM_SHARED((_N, feat), jnp.float32)]
            + [pltpu.SemaphoreType.DMA for _ in range(2 * ring)]
        ),
    )
    def sc(x_hbm, src_hbm, dst_hbm, out_hbm, sidx_big, didx_big, *bufs):
        rows = bufs[:ring]
        acc = bufs[ring]
        semg = bufs[ring + 1:2 * ring + 1]
        sems = bufs[2 * ring + 1:3 * ring + 1]
        c = lax.axis_index("c")
        s = lax.axis_index("s")
        wid = s * _NC + c

        def zrow(i, _):
            def zcol(j, _):
                rows[0][i, pl.ds(j * _L, _L)] = jnp.zeros((_L,), jnp.float32)
                return 0
            return lax.fori_loop(0, feat // _L, zcol, 0)
        lax.fori_loop(0, _CHUNK, zrow, 0)

        r0 = s * base_rows
        nz = jnp.where(s == _NS - 1, zfull_l, zfull)

        def zacc(i, _):
            pltpu.sync_copy(rows[0], acc.at[pl.ds(r0 + i * _CHUNK, _CHUNK)])
            return 0
        lax.fori_loop(0, nz, zacc, 0)
        if ztail:
            @pl.when(s < _NS - 1)
            def _():
                pltpu.sync_copy(rows[0].at[pl.ds(0, ztail)],
                                acc.at[pl.ds(r0 + zfull * _CHUNK, ztail)])
        if ztail_l:
            @pl.when(s == _NS - 1)
            def _():
                pltpu.sync_copy(rows[0].at[pl.ds(0, ztail_l)],
                                acc.at[pl.ds(r0 + zfull_l * _CHUNK, ztail_l)])
        plsc.subcore_barrier()

        def fire_gather(b, j):
            pltpu.async_copy(x_hbm.at[sidx_big.at[j]], rows[b], semg[b])

        def wait_gather(b):
            pltpu.make_async_copy(x_hbm.at[sidx_big.at[0]], rows[b],
                                  semg[b]).wait()

        def fire_scatter(b, j):
            pltpu.async_copy(rows[b], acc.at[didx_big.at[j]], sems[b],
                             add=True)

        def wait_scatter(b):
            pltpu.make_async_copy(rows[b], acc.at[didx_big.at[0]],
                                  sems[b]).wait()

        cw0 = wid * _CPW
        for bi in range(nblk):
            cb = cw0 + bi * iblk
            pltpu.sync_copy(src_hbm.at[pl.ds(cb, iblk)], sidx_big)
            pltpu.sync_copy(dst_hbm.at[pl.ds(cb, iblk)], didx_big)
            for b in range(pf):
                fire_gather(b, b)

            def step(jj, _):
                for b in range(ring):
                    j = jj * ring + b
                    jn = j + pf
                    bn = (b + pf) % ring

                    @pl.when(jn < iblk)
                    def _():
                        @pl.when(j >= ring - pf)
                        def _():
                            wait_scatter(bn)
                        fire_gather(bn, jn)
                    wait_gather(b)
                    fire_scatter(b, j)
                return 0
            lax.fori_loop(0, iblk // ring, step, 0)
            for b in range(ring):
                wait_scatter(b)

        @pl.when(wid < _XTRA)
        def _():
            xc = _NW * _CPW + wid
            pltpu.sync_copy(src_hbm.at[pl.ds(xc, 1)],
                            sidx_big.at[pl.ds(0, 1)])
            pltpu.sync_copy(dst_hbm.at[pl.ds(xc, 1)],
                            didx_big.at[pl.ds(0, 1)])
            fire_gather(0, 0)
            wait_gather(0)
            fire_scatter(0, 0)
            wait_scatter(0)

        plsc.subcore_barrier()

        @pl.when(s < _NS - 1)
        def _():
            pltpu.sync_copy(acc.at[pl.ds(r0, base_rows)],
                            out_hbm.at[c, pl.ds(r0, base_rows)])

        @pl.when(s == _NS - 1)
        def _():
            pltpu.sync_copy(acc.at[pl.ds(r0, last_rows)],
                            out_hbm.at[c, pl.ds(r0, last_rows)])

    return sc


_sc_scatter = {f: _make_sc_scatter(f) for f in (128, 112, 96, 64)}

_R = 2000
_NB = _N // _R


def _make_layer(fins, hmid, hout, outs):
    k = len(fins)
    covered = sum(outs)

    def body(*refs):
        x_refs = refs[0:k]
        agg_refs = refs[k:2 * k]
        W1_refs = refs[2 * k:3 * k]
        b1_ref, W2_ref, b2_ref, g_ref, be_ref = refs[3 * k:3 * k + 5]
        o_refs = refs[3 * k + 5:3 * k + 5 + len(outs)]
        h_all, stats = refs[3 * k + 5 + len(outs):]
        i = pl.program_id(0)

        @pl.when(i == 0)
        def _():
            stats[...] = jnp.zeros_like(stats)

        @pl.when(i < _NB)
        def _():
            a1 = b1_ref[...] + jnp.zeros((_R, hmid), jnp.float32)
            for xr, ar, wr in zip(x_refs, agg_refs, W1_refs):
                t = xr[...] + ar[0] + ar[1]
                a1 = a1 + jnp.dot(t, wr[...],
                                  preferred_element_type=jnp.float32)
            a1 = jnp.maximum(a1, 0.0)
            a2 = jnp.dot(a1, W2_ref[...], preferred_element_type=jnp.float32)
            a2 = jnp.maximum(a2 + b2_ref[...], 0.0)
            h_all[pl.ds(i * _R, _R)] = a2
            stats[0:1, :] = stats[0:1, :] + jnp.sum(a2, axis=0, keepdims=True)
            stats[1:2, :] = stats[1:2, :] + jnp.sum(a2 * a2, axis=0,
                                                    keepdims=True)

        @pl.when(i == _NB)
        def _():
            m = stats[0:1, :] / _N
            v = stats[1:2, :] / _N - m * m
            a = g_ref[...] / jnp.sqrt(v + 1e-5)
            stats[2:3, :] = a
            stats[3:4, :] = be_ref[...] - m * a

        @pl.when(i >= _NB)
        def _():
            j = i - _NB
            hb = h_all[pl.ds(j * _R, _R)] * stats[2:3, :] + stats[3:4, :]
            col = 0
            for o_ref, w in zip(o_refs, outs):
                if col + w <= hout:
                    o_ref[...] = hb[:, col:col + w]
                else:
                    o_ref[...] = jnp.concatenate(
                        [hb[:, col:hout],
                         jnp.zeros((_R, col + w - hout), jnp.float32)], axis=1)
                col += w

    in_specs = (
        [pl.BlockSpec((_R, f), lambda i: (i % _NB, 0)) for f in fins]
        + [pl.BlockSpec((2, _R, f), lambda i: (0, i % _NB, 0)) for f in fins]
        + [pl.BlockSpec((f, hmid), lambda i: (0, 0)) for f in fins]
        + [pl.BlockSpec((1, hmid), lambda i: (0, 0)),
           pl.BlockSpec((hmid, hout), lambda i: (0, 0)),
           pl.BlockSpec((1, hout), lambda i: (0, 0)),
           pl.BlockSpec((1, hout), lambda i: (0, 0)),
           pl.BlockSpec((1, hout), lambda i: (0, 0))]
    )
    return pl.pallas_call(
        body,
        grid=(2 * _NB,),
        in_specs=in_specs,
        out_specs=[pl.BlockSpec((_R, w), lambda i: (i % _NB, 0))
                   for w in outs],
        out_shape=[jax.ShapeDtypeStruct((_N, w), jnp.float32) for w in outs],
        scratch_shapes=[
            pltpu.VMEM((_N, hout), jnp.float32),
            pltpu.VMEM((8, hout), jnp.float32),
        ],
    )


_layer1 = _make_layer([128], 198, 198, [112, 96])
_layer2 = _make_layer([112, 96], 64, 64, [64])
_layer3 = _make_layer([64], 32, 32, [32])


def _head_body(h_ref, b_ref, fW1_ref, fb1_ref, fW2_ref, fb2_ref,
               oW_ref, ob_ref, o_ref, pacc):
    i = pl.program_id(0)

    @pl.when(i == 0)
    def _():
        pacc[...] = jnp.zeros_like(pacc)

    b = b_ref[0]
    onehot = (lax.broadcasted_iota(jnp.int32, (_G, 1), 0) == b)
    onehot = onehot.astype(jnp.float32)
    pacc[...] = pacc[...] + lax.dot_general(
        onehot, h_ref[...], (((1,), (0,)), ((), ())),
        preferred_element_type=jnp.float32,
        precision=lax.Precision.HIGHEST)

    @pl.when(i == _NB - 1)
    def _():
        p = pacc[...]
        p = jnp.maximum(
            jnp.dot(p, fW1_ref[...], preferred_element_type=jnp.float32)
            + fb1_ref[...], 0.0)
        p = jnp.maximum(
            jnp.dot(p, fW2_ref[...], preferred_element_type=jnp.float32)
            + fb2_ref[...], 0.0)
        o_ref[...] = jnp.dot(
            p, oW_ref[...], preferred_element_type=jnp.float32) + ob_ref[...]


_head = pl.pallas_call(
    _head_body,
    grid=(_NB,),
    in_specs=[
        pl.BlockSpec((_R, 32), lambda i: (i, 0)),
        pl.BlockSpec((1, 1, _R), lambda i: (i, 0, 0)),
        pl.BlockSpec((32, 16), lambda i: (0, 0)),
        pl.BlockSpec((1, 16), lambda i: (0, 0)),
        pl.BlockSpec((16, 8), lambda i: (0, 0)),
        pl.BlockSpec((1, 8), lambda i: (0, 0)),
        pl.BlockSpec((8, 2), lambda i: (0, 0)),
        pl.BlockSpec((1, 2), lambda i: (0, 0)),
    ],
    out_specs=pl.BlockSpec((_G, 2), lambda i: (0, 0)),
    out_shape=jax.ShapeDtypeStruct((_G, 2), jnp.float32),
    scratch_shapes=[pltpu.VMEM((_G, 32), jnp.float32)],
)


def kernel(x, edge_index, batch,
           W1, b1, W2, b2, g1, be1,
           W3, b3, W4, b4, g2, be2,
           W5, b5, W6, b6, g3, be3,
           fW1, fb1, fW2, fb2, oW, ob):
    src = edge_index[0].reshape(_NCHUNK, _CHUNK)
    dst = edge_index[1].reshape(_NCHUNK, _CHUNK)
    xp = jnp.pad(x, ((0, 0), (0, 128 - 114)))
    W1p = jnp.pad(W1, ((0, 128 - 114), (0, 0)))
    W3a = W3[:112]
    W3b = jnp.pad(W3[112:], ((0, 96 - (198 - 112)), (0, 0)))

    agg0 = _sc_scatter[128](xp, src, dst)
    h1a, h1b = _layer1(xp, agg0, W1p, b1[None], W2, b2[None],
                       g1[None], be1[None])
    aggA = _sc_scatter[112](h1a, src, dst)
    aggB = _sc_scatter[96](h1b, src, dst)
    h2, = _layer2(h1a, h1b, aggA, aggB, W3a, W3b, b3[None], W4, b4[None],
                  g2[None], be2[None])
    agg2 = _sc_scatter[64](h2, src, dst)
    h3, = _layer3(h2, agg2, W5, b5[None], W6, b6[None], g3[None], be3[None])
    batch3 = batch.reshape(_NB, 1, _R)
    return _head(h3, batch3, fW1, fb1[None], fW2, fb2[None], oW, ob[None])
